# Initial kernel scaffold; baseline (speedup 1.0000x reference)
import jax
import jax.numpy as jnp
from jax.experimental import pallas as pl


def kernel(input_pc, params):
    raise NotImplementedError("write your pallas kernel here")


# speedup vs baseline: 1.0005x; 1.0005x over previous
import functools
import jax, jax.numpy as jnp
import numpy as np
from jax.experimental import pallas as pl
from jax.experimental.pallas import tpu as pltpu

EPS = 1e-5
INV = np.float32(1.0 / np.sqrt(1.0 + EPS))


def _square_distance(src, dst):
    s2 = jnp.sum(src ** 2, -1)[:, :, None]
    d2 = jnp.sum(dst ** 2, -1)[:, None, :]
    return s2 + d2 - 2.0 * jnp.einsum('bnc,bmc->bnm', src, dst)


def _index_points(points, idx):
    B = points.shape[0]
    bidx = jnp.arange(B).reshape((B,) + (1,) * (idx.ndim - 1))
    return points[bidx, idx]


def _fps(xyz, npoint):
    B, N, _ = xyz.shape
    def step(state, _):
        distance, farthest = state
        centroid = jnp.take_along_axis(xyz, farthest[:, None, None], axis=1)
        d = jnp.sum((xyz - centroid) ** 2, -1)
        distance = jnp.minimum(distance, d)
        new_far = jnp.argmax(distance, -1).astype(jnp.int32)
        return (distance, new_far), farthest
    init = (jnp.full((B, N), 1e10, dtype=xyz.dtype), jnp.zeros((B,), dtype=jnp.int32))
    _, idxs = jax.lax.scan(step, init, None, length=npoint)
    return jnp.transpose(idxs)


def _query_ball(radius, nsample, xyz, new_xyz):
    B, N, _ = xyz.shape
    S = new_xyz.shape[1]
    sqr = _square_distance(new_xyz, xyz)
    ar = jnp.broadcast_to(jnp.arange(N, dtype=jnp.int32), (B, S, N))
    gidx = jnp.where(sqr > radius ** 2, N, ar)
    gidx = jnp.sort(gidx, axis=-1)[:, :, :nsample]
    first = gidx[:, :, :1]
    return jnp.where(gidx == N, first, gidx)


def _apply_mlp(x, layers):
    for p in layers:
        x = x @ p['W'] + p['b']
        x = jax.nn.relu(p['g'] * (x * INV) + p['be'])
    return x


def _sa_msg(xyz, points, npoint, radii, nsamples, branches):
    fi = _fps(xyz, npoint)
    new_xyz = _index_points(xyz, fi)
    outs = []
    for r, K, layers in zip(radii, nsamples, branches):
        gidx = _query_ball(r, K, xyz, new_xyz)
        gxyz = _index_points(xyz, gidx) - new_xyz[:, :, None, :]
        if points is None:
            gp = gxyz
        else:
            gp = jnp.concatenate([_index_points(points, gidx), gxyz], -1)
        gp = _apply_mlp(gp, layers)
        outs.append(jnp.max(gp, axis=2))
    return new_xyz, jnp.concatenate(outs, -1)


def _feature_prop(xyz1, xyz2, points1, points2, layers):
    d = _square_distance(xyz1, xyz2)
    negv, idx = jax.lax.top_k(-d, 3)
    dists = -negv
    w = 1.0 / (dists + 1e-8)
    w = w / jnp.sum(w, -1, keepdims=True)
    interp = jnp.sum(_index_points(points2, idx) * w[..., None], axis=2)
    x = interp if points1 is None else jnp.concatenate([points1, interp], -1)
    return _apply_mlp(x, layers)


def _head_kernel(x_ref, w1_ref, b1_ref, g1_ref, be1_ref, w2_ref, b2_ref, o_ref):
    x = x_ref[...]
    h = jnp.dot(x, w1_ref[...], preferred_element_type=jnp.float32) + b1_ref[...]
    h = jax.nn.relu(g1_ref[...] * (h * INV) + be1_ref[...])
    o_ref[...] = jnp.dot(h, w2_ref[...], preferred_element_type=jnp.float32) + b2_ref[...]


def _head(l0p, h):
    B, N, C = l0p.shape
    x = l0p.reshape(B * N, C)
    M = B * N
    MT = 2048
    W2p = jnp.zeros((128, 128), jnp.float32).at[:, :13].set(h['W2'])
    b2p = jnp.zeros((128,), jnp.float32).at[:13].set(h['b2'])
    out = pl.pallas_call(
        _head_kernel,
        grid=(M // MT,),
        in_specs=[
            pl.BlockSpec((MT, C), lambda i: (i, 0)),
            pl.BlockSpec((C, 128), lambda i: (0, 0)),
            pl.BlockSpec((128,), lambda i: (0,)),
            pl.BlockSpec((128,), lambda i: (0,)),
            pl.BlockSpec((128,), lambda i: (0,)),
            pl.BlockSpec((128, 128), lambda i: (0, 0)),
            pl.BlockSpec((128,), lambda i: (0,)),
        ],
        out_specs=pl.BlockSpec((MT, 128), lambda i: (i, 0)),
        out_shape=jax.ShapeDtypeStruct((M, 128), jnp.float32),
    )(x, h['W1'], h['b1'], h['g1'], h['be1'], W2p, b2p)
    return out[:, :13].reshape(B, N, 13)


def kernel(input_pc, params):
    xyz0 = input_pc[:, :, :3]
    feat0 = input_pc[:, :, 3:]
    l1x, l1p = _sa_msg(xyz0, feat0, 1024, [1.0, 3.0], [8, 32], params['sa'][0])
    l2x, l2p = _sa_msg(l1x, l1p, 512, [2.0, 4.0], [8, 32], params['sa'][1])
    l3x, l3p = _sa_msg(l2x, l2p, 256, [3.0, 6.0], [16, 32], params['sa'][2])
    l4x, l4p = _sa_msg(l3x, l3p, 128, [4.0, 8.0], [16, 32], params['sa'][3])
    l3p = _feature_prop(l3x, l4x, l3p, l4p, params['fp'][0])
    l2p = _feature_prop(l2x, l3x, l2p, l3p, params['fp'][1])
    l1p = _feature_prop(l1x, l2x, l1p, l2p, params['fp'][2])
    l0p = _feature_prop(xyz0, l1x, None, l1p, params['fp'][3])
    return _head(l0p, params['head'])


# speedup vs baseline: 16.4364x; 16.4283x over previous
import functools
import jax
import jax.numpy as jnp
import numpy as np
from jax import lax
from jax.experimental import pallas as pl
from jax.experimental.pallas import tpu as pltpu
from jax.experimental.pallas import tpu_sc as plsc

EPS = 1e-5
INV = np.float32(1.0 / np.sqrt(1.0 + EPS))
SC_NC = 2
SC_NS = 16
SC_W = SC_NC * SC_NS


def _div_block(n, target):
    d = min(n, target)
    while n % d:
        d -= 1
    return d


def _fps_body(xp_ref, yp_ref, zp_ref, nx_ref, ny_ref, nz_ref, *, npoint):
    xp = xp_ref[...]
    yp = yp_ref[...]
    zp = zp_ref[...]
    B, N = xp.shape
    iota = lax.broadcasted_iota(jnp.int32, (B, N), 1)
    iota_s = lax.broadcasted_iota(jnp.int32, (B, npoint), 1)

    def step(t, carry):
        dist, far, ax, ay, az = carry
        onehot = iota == far
        cx = jnp.sum(jnp.where(onehot, xp, 0.0), axis=1, keepdims=True)
        cy = jnp.sum(jnp.where(onehot, yp, 0.0), axis=1, keepdims=True)
        cz = jnp.sum(jnp.where(onehot, zp, 0.0), axis=1, keepdims=True)
        sel = iota_s == t
        ax = jnp.where(sel, cx, ax)
        ay = jnp.where(sel, cy, ay)
        az = jnp.where(sel, cz, az)
        dx = xp - cx
        dy = yp - cy
        dz = zp - cz
        d = (dx * dx + dy * dy) + dz * dz
        dist = jnp.minimum(dist, d)
        m = jnp.max(dist, axis=1, keepdims=True)
        far = jnp.min(jnp.where(dist == m, iota, N), axis=1, keepdims=True)
        return dist, far, ax, ay, az

    zs = jnp.zeros((B, npoint), jnp.float32)
    init = (jnp.full((B, N), 1e10, jnp.float32),
            jnp.zeros((B, 1), jnp.int32), zs, zs, zs)
    _, _, ax, ay, az = lax.fori_loop(0, npoint, step, init)
    nx_ref[...] = ax
    ny_ref[...] = ay
    nz_ref[...] = az


def _fps(xyz, npoint):
    B, N, _ = xyz.shape
    xt = jnp.transpose(xyz, (0, 2, 1))
    outs = pl.pallas_call(
        functools.partial(_fps_body, npoint=npoint),
        in_specs=[pl.BlockSpec((B, N), lambda: (0, 0))] * 3,
        out_specs=[pl.BlockSpec((B, npoint), lambda: (0, 0))] * 3,
        out_shape=[jax.ShapeDtypeStruct((B, npoint), jnp.float32)] * 3,
    )(xt[:, 0], xt[:, 1], xt[:, 2])
    return jnp.stack(outs, axis=-1)


def _dist_block(a, b2):
    s2 = jnp.sum(a * a, axis=1, keepdims=True)
    d2 = jnp.sum(b2 * b2, axis=1, keepdims=True).reshape(1, -1)
    cr = lax.dot_general(a, b2, (((1,), (1,)), ((), ())),
                         preferred_element_type=jnp.float32)
    return (s2 + d2) - 2.0 * cr


def _dist_body(a_ref, b2_ref, d_ref):
    d_ref[0] = _dist_block(a_ref[0], b2_ref[0])


def _sqdist(src, dst):
    B, S, _ = src.shape
    N = dst.shape[1]
    SB = _div_block(S, 256)
    return pl.pallas_call(
        _dist_body,
        grid=(B, S // SB),
        in_specs=[
            pl.BlockSpec((1, SB, 3), lambda b, s: (b, s, 0)),
            pl.BlockSpec((1, N, 3), lambda b, s: (b, 0, 0)),
        ],
        out_specs=pl.BlockSpec((1, SB, N), lambda b, s: (b, s, 0)),
        out_shape=jax.ShapeDtypeStruct((B, S, N), jnp.float32),
    )(src, dst)


def _sa_tail_body(g_ref, q_ref, w1_ref, b1_ref, g1_ref, be1_ref,
                  w2_ref, b2_ref, g2_ref, be2_ref,
                  w3_ref, b3_ref, g3_ref, be3_ref, o_ref):
    MT, K, Cp = g_ref.shape
    x = g_ref[...] - q_ref[...][:, None, :]
    x = x.reshape(MT * K, Cp)
    x = jnp.dot(x, w1_ref[...], preferred_element_type=jnp.float32) + b1_ref[...]
    x = jax.nn.relu(g1_ref[...] * (x * INV) + be1_ref[...])
    x = jnp.dot(x, w2_ref[...], preferred_element_type=jnp.float32) + b2_ref[...]
    x = jax.nn.relu(g2_ref[...] * (x * INV) + be2_ref[...])
    x = jnp.dot(x, w3_ref[...], preferred_element_type=jnp.float32) + b3_ref[...]
    x = jax.nn.relu(g3_ref[...] * (x * INV) + be3_ref[...])
    C3 = x.shape[1]
    o_ref[...] = jnp.max(x.reshape(MT, K, C3), axis=1)


def _sa_tail(gath, sub, W1p, layers):
    M, K, Cp = gath.shape
    l1, l2, l3 = layers
    C1 = l1['W'].shape[1]
    C2 = l2['W'].shape[1]
    C3 = l3['W'].shape[1]
    MT = _div_block(M, 128)
    return pl.pallas_call(
        _sa_tail_body,
        grid=(M // MT,),
        in_specs=[
            pl.BlockSpec((MT, K, Cp), lambda i: (i, 0, 0)),
            pl.BlockSpec((MT, Cp), lambda i: (i, 0)),
            pl.BlockSpec((Cp, C1), lambda i: (0, 0)),
            pl.BlockSpec((C1,), lambda i: (0,)),
            pl.BlockSpec((C1,), lambda i: (0,)),
            pl.BlockSpec((C1,), lambda i: (0,)),
            pl.BlockSpec((C1, C2), lambda i: (0, 0)),
            pl.BlockSpec((C2,), lambda i: (0,)),
            pl.BlockSpec((C2,), lambda i: (0,)),
            pl.BlockSpec((C2,), lambda i: (0,)),
            pl.BlockSpec((C2, C3), lambda i: (0, 0)),
            pl.BlockSpec((C3,), lambda i: (0,)),
            pl.BlockSpec((C3,), lambda i: (0,)),
            pl.BlockSpec((C3,), lambda i: (0,)),
        ],
        out_specs=pl.BlockSpec((MT, C3), lambda i: (i, 0)),
        out_shape=jax.ShapeDtypeStruct((M, C3), jnp.float32),
    )(gath, sub, W1p, l1['b'], l1['g'], l1['be'],
      l2['W'], l2['b'], l2['g'], l2['be'],
      l3['W'], l3['b'], l3['g'], l3['be'])


def _knn3_body(a_ref, b2_ref, i_ref, w_ref, *, S2):
    a = a_ref[0]
    SB = a.shape[0]
    D = _dist_block(a, b2_ref[0])
    iota = lax.broadcasted_iota(jnp.int32, (SB, S2), 1)
    vals, idxs = [], []
    for _ in range(3):
        m = jnp.min(D, axis=1, keepdims=True)
        am = jnp.min(jnp.where(D == m, iota, S2), axis=1, keepdims=True)
        vals.append(m)
        idxs.append(am)
        D = jnp.where(iota == am, jnp.float32(np.inf), D)
    b = pl.program_id(0)
    i_ref[0] = jnp.concatenate(idxs, axis=1) + b * S2
    d3 = jnp.concatenate(vals, axis=1)
    w = 1.0 / (d3 + 1e-8)
    w_ref[0] = w / jnp.sum(w, axis=1, keepdims=True)


def _knn3(xyz1, xyz2):
    B, S1, _ = xyz1.shape
    S2 = xyz2.shape[1]
    SB = _div_block(S1, 512)
    return pl.pallas_call(
        functools.partial(_knn3_body, S2=S2),
        grid=(B, S1 // SB),
        in_specs=[
            pl.BlockSpec((1, SB, 3), lambda b, s: (b, s, 0)),
            pl.BlockSpec((1, S2, 3), lambda b, s: (b, 0, 0)),
        ],
        out_specs=[
            pl.BlockSpec((1, SB, 3), lambda b, s: (b, s, 0)),
            pl.BlockSpec((1, SB, 3), lambda b, s: (b, s, 0)),
        ],
        out_shape=[
            jax.ShapeDtypeStruct((B, S1, 3), jnp.int32),
            jax.ShapeDtypeStruct((B, S1, 3), jnp.float32),
        ],
    )(xyz1, xyz2)


def _fp_tail_body(*refs, n_layers, has_p1, has_head):
    it = iter(refs)
    g_ref = next(it)
    w_ref = next(it)
    p1_ref = next(it) if has_p1 else None
    lw = [(next(it), next(it), next(it), next(it)) for _ in range(n_layers)]
    if has_head:
        hw1, hb1, hg1, hbe1, hw2, hb2 = (next(it) for _ in range(6))
    o_ref = next(it)

    g = g_ref[...]
    wv = w_ref[...]
    interp = (g[:, 0, :] * wv[:, 0:1] + g[:, 1, :] * wv[:, 1:2]) \
        + g[:, 2, :] * wv[:, 2:3]
    if has_p1:
        x = jnp.concatenate([p1_ref[...], interp], axis=-1)
    else:
        x = interp
    for (W, bb, gg, be) in lw:
        x = jnp.dot(x, W[...], preferred_element_type=jnp.float32) + bb[...]
        x = jax.nn.relu(gg[...] * (x * INV) + be[...])
    if has_head:
        x = jnp.dot(x, hw1[...], preferred_element_type=jnp.float32) + hb1[...]
        x = jax.nn.relu(hg1[...] * (x * INV) + hbe1[...])
        x = jnp.dot(x, hw2[...], preferred_element_type=jnp.float32) + hb2[...]
    o_ref[...] = x


def _fp_tail(gath, w, p1, layers, head=None):
    M, _, C2 = gath.shape
    MT = _div_block(M, 512)
    has_p1 = p1 is not None
    has_head = head is not None
    n_layers = len(layers)

    args = [gath, w]
    in_specs = [
        pl.BlockSpec((MT, 3, C2), lambda i: (i, 0, 0)),
        pl.BlockSpec((MT, 3), lambda i: (i, 0)),
    ]
    cin = C2
    if has_p1:
        C1p = p1.shape[1]
        cin += C1p
        args.append(p1)
        in_specs.append(pl.BlockSpec((MT, C1p), lambda i: (i, 0)))
    c = cin
    for p in layers:
        co = p['W'].shape[1]
        args += [p['W'], p['b'], p['g'], p['be']]
        in_specs += [
            pl.BlockSpec((c, co), lambda i: (0, 0)),
            pl.BlockSpec((co,), lambda i: (0,)),
            pl.BlockSpec((co,), lambda i: (0,)),
            pl.BlockSpec((co,), lambda i: (0,)),
        ]
        c = co
    if has_head:
        hW2, hb2 = head['W2p'], head['b2p']
        co2 = hW2.shape[1]
        args += [head['W1'], head['b1'], head['g1'], head['be1'], hW2, hb2]
        in_specs += [
            pl.BlockSpec((c, 128), lambda i: (0, 0)),
            pl.BlockSpec((128,), lambda i: (0,)),
            pl.BlockSpec((128,), lambda i: (0,)),
            pl.BlockSpec((128,), lambda i: (0,)),
            pl.BlockSpec((128, co2), lambda i: (0, 0)),
            pl.BlockSpec((co2,), lambda i: (0,)),
        ]
        c = co2
    return pl.pallas_call(
        functools.partial(_fp_tail_body, n_layers=n_layers, has_p1=has_p1,
                          has_head=has_head),
        grid=(M // MT,),
        in_specs=in_specs,
        out_specs=pl.BlockSpec((MT, c), lambda i: (i, 0)),
        out_shape=jax.ShapeDtypeStruct((M, c), jnp.float32),
    )(*args)


def _sc_compact_call(D, Nsrc, K1, K2, r1sq, r2sq, B):
    R, N = D.shape
    RW = R // SC_W
    CH = _div_block(RW, 8)
    WPB = SC_W // B
    K1p, K2p = K1 + 16, K2 + 16
    nchunks = N // 16
    r1c = np.float32(r1sq)
    r2c = np.float32(r2sq)
    mesh = plsc.VectorSubcoreMesh(core_axis_name="c", subcore_axis_name="s")

    @functools.partial(
        pl.kernel, mesh=mesh,
        out_type=[
            jax.ShapeDtypeStruct((R, K1p), jnp.int32),
            jax.ShapeDtypeStruct((R, K2p), jnp.int32),
        ],
        scratch_types=[
            pltpu.VMEM((CH, N), jnp.float32),
            pltpu.VMEM((CH, K1p), jnp.int32),
            pltpu.VMEM((CH, K2p), jnp.int32),
        ],
        compiler_params=pltpu.CompilerParams(
            use_tc_tiling_on_sc=False, needs_layout_passes=False),
    )
    def compact(d_hbm, g1_hbm, g2_hbm, dbuf, g1buf, g2buf):
        wid = lax.axis_index("s") * SC_NC + lax.axis_index("c")
        base_pt = (wid // WPB) * N
        row0 = wid * RW

        def outer(i, _):
            rb = row0 + i * CH
            pltpu.sync_copy(d_hbm.at[pl.ds(rb, CH)], dbuf)
            for r in range(CH):
                def cond(carry):
                    c, c1, c2, f1, f2 = carry
                    return (c < nchunks) & ((c1 < K1) | (c2 < K2))

                def body(carry):
                    c, c1, c2, f1, f2 = carry
                    dv = dbuf[r, pl.ds(c * 16, 16)]
                    gi = lax.iota(jnp.int32, 16) + (base_pt + c * 16)
                    m1 = dv <= r1c
                    m2 = dv <= r2c
                    pc1 = plsc.all_reduce_population_count(m1)[0]
                    pc2 = plsc.all_reduce_population_count(m2)[0]
                    off1 = jnp.minimum(c1, K1)
                    off2 = jnp.minimum(c2, K2)
                    plsc.store_compressed(
                        g1buf.at[r].at[pl.ds(off1, 16)], gi, mask=m1)
                    plsc.store_compressed(
                        g2buf.at[r].at[pl.ds(off2, 16)], gi, mask=m2)

                    fm1 = base_pt + c * 16 + plsc.all_reduce_ffs(m1)[0]
                    fm2 = base_pt + c * 16 + plsc.all_reduce_ffs(m2)[0]
                    f1 = jnp.where((c1 == 0) & (pc1 > 0), fm1, f1)
                    f2 = jnp.where((c2 == 0) & (pc2 > 0), fm2, f2)
                    c1 = jnp.where(c1 < K1, c1 + pc1, c1)
                    c2 = jnp.where(c2 < K2, c2 + pc2, c2)
                    return (c + 1, c1, c2, f1, f2)

                last = jnp.int32(base_pt + N - 1)
                init = (jnp.int32(0), jnp.int32(0), jnp.int32(0), last, last)
                _, c1, c2, f1, f2 = lax.while_loop(cond, body, init)
                for j in range(max(1, K1 // 16)):
                    sl = lax.iota(jnp.int32, 16) + j * 16
                    cur = g1buf[r, pl.ds(j * 16, 16)]
                    g1buf[r, pl.ds(j * 16, 16)] = jnp.where(sl < c1, cur, f1)
                for j in range(max(1, K2 // 16)):
                    sl = lax.iota(jnp.int32, 16) + j * 16
                    cur = g2buf[r, pl.ds(j * 16, 16)]
                    g2buf[r, pl.ds(j * 16, 16)] = jnp.where(sl < c2, cur, f2)
            pltpu.sync_copy(g1buf, g1_hbm.at[pl.ds(rb, CH)])
            pltpu.sync_copy(g2buf, g2_hbm.at[pl.ds(rb, CH)])
            return 0

        lax.fori_loop(0, RW // CH, outer, 0)

    g1p, g2p = compact(D)
    return g1p[:, :K1], g2p[:, :K2]


def _sc_gather_call(table, idx):
    V, C = table.shape
    R = idx.shape[0]
    RW = R // SC_W
    CR = 128
    while RW % CR or CR * C * 4 > 262144:
        CR //= 2
    mesh = plsc.VectorSubcoreMesh(core_axis_name="c", subcore_axis_name="s")

    @functools.partial(
        pl.kernel, mesh=mesh,
        out_type=jax.ShapeDtypeStruct((R, C), jnp.float32),
        scratch_types=[
            pltpu.VMEM((CR,), jnp.int32),
            pltpu.VMEM((CR, C), jnp.float32),
            pltpu.SemaphoreType.DMA,
        ],
        compiler_params=pltpu.CompilerParams(use_tc_tiling_on_sc=False),
    )
    def gather(tab_hbm, idx_hbm, out_hbm, idxv, rows, sem):
        wid = lax.axis_index("s") * SC_NC + lax.axis_index("c")

        def body(i, _):
            base = wid * RW + i * CR
            pltpu.sync_copy(idx_hbm.at[pl.ds(base, CR)], idxv)
            pltpu.async_copy(tab_hbm.at[idxv], rows, sem).wait()
            pltpu.sync_copy(rows, out_hbm.at[pl.ds(base, CR)])
            return 0

        lax.fori_loop(0, RW // CR, body, 0)

    return gather(table, idx)


def _sa_msg_layer(xyz, points, npoint, radii, nsamples, branches):
    B, N, _ = xyz.shape
    new_xyz = _fps(xyz, npoint)
    D = _sqdist(new_xyz, xyz).reshape(B * npoint, N)
    (r1, r2), (K1, K2) = radii, nsamples
    g1, g2 = _sc_compact_call(D, N, K1, K2, r1 * r1, r2 * r2, B)

    C = points.shape[-1]
    Cin = C + 3
    Cp = ((Cin + 15) // 16) * 16
    M = B * npoint
    X = jnp.concatenate([points, xyz], axis=-1).reshape(B * N, Cin)
    Xp = jnp.pad(X, ((0, 0), (0, Cp - Cin)))
    sub = jnp.pad(new_xyz.reshape(M, 3), ((0, 0), (C, Cp - Cin)))
    outs = []
    for (gidx, K, layers) in ((g1, K1, branches[0]), (g2, K2, branches[1])):
        W1p = jnp.pad(layers[0]['W'], ((0, Cp - Cin), (0, 0)))
        G = _sc_gather_call(Xp, gidx.reshape(-1))
        G = G.reshape(M, K, Cp)
        outs.append(_sa_tail(G, sub, W1p, layers))
    new_points = jnp.concatenate(outs, axis=-1).reshape(B, npoint, -1)
    return new_xyz, new_points


def _fp_layer(xyz1, xyz2, points1, points2, layers, head=None):
    B, S1, _ = xyz1.shape
    S2 = xyz2.shape[1]
    C2 = points2.shape[-1]
    idx, w = _knn3(xyz1, xyz2)
    G = _sc_gather_call(points2.reshape(B * S2, C2), idx.reshape(-1))
    G = G.reshape(B * S1, 3, C2)
    wf = w.reshape(B * S1, 3)
    p1 = None if points1 is None else points1.reshape(B * S1, -1)
    out = _fp_tail(G, wf, p1, layers, head=head)
    return out.reshape(B, S1, out.shape[-1])


def _forward(input_pc, params, sa_cfgs, num_classes):
    B = input_pc.shape[0]
    xyz0 = input_pc[:, :, :3]
    feat0 = input_pc[:, :, 3:]
    h = params['head']
    head = dict(W1=h['W1'], b1=h['b1'], g1=h['g1'], be1=h['be1'],
                W2p=jnp.zeros((128, 128), jnp.float32).at[:, :num_classes].set(h['W2']),
                b2p=jnp.zeros((128,), jnp.float32).at[:num_classes].set(h['b2']))

    xs, ps = [xyz0], [feat0]
    x, p = xyz0, feat0
    for cfg, br in zip(sa_cfgs, params['sa']):
        npoint, radii, nsamples = cfg[0], cfg[1], cfg[2]
        x, p = _sa_msg_layer(x, p, npoint, radii, nsamples, br)
        xs.append(x)
        ps.append(p)

    n = len(xs) - 1
    for i in range(n - 1):
        ps[n - 1 - i] = _fp_layer(xs[n - 1 - i], xs[n - i], ps[n - 1 - i],
                                  ps[n - i], params['fp'][i])
    out = _fp_layer(xs[0], xs[1], None, ps[1], params['fp'][n - 1], head=head)
    return out[:, :, :num_classes]


_SA_CFGS = [
    (1024, [1.0, 3.0], [8, 32]),
    (512, [2.0, 4.0], [8, 32]),
    (256, [3.0, 6.0], [16, 32]),
    (128, [4.0, 8.0], [16, 32]),
]


def kernel(input_pc, params):
    return _forward(input_pc, params, _SA_CFGS, 13)


# speedup vs baseline: 16.6378x; 1.0123x over previous
import functools
import jax
import jax.numpy as jnp
import numpy as np
from jax import lax
from jax.experimental import pallas as pl
from jax.experimental.pallas import tpu as pltpu
from jax.experimental.pallas import tpu_sc as plsc

EPS = 1e-5
INV = np.float32(1.0 / np.sqrt(1.0 + EPS))
SC_NC = 2
SC_NS = 16
SC_W = SC_NC * SC_NS


def _div_block(n, target):
    d = min(n, target)
    while n % d:
        d -= 1
    return d


def _fps_body(xp_ref, yp_ref, zp_ref, nx_ref, ny_ref, nz_ref, *, npoint):
    xp = xp_ref[...]
    yp = yp_ref[...]
    zp = zp_ref[...]
    B, N = xp.shape
    iota = lax.broadcasted_iota(jnp.int32, (B, N), 1)
    iota_s = lax.broadcasted_iota(jnp.int32, (B, npoint), 1)

    def step(t, carry):
        dist, far, ax, ay, az = carry
        onehot = iota == far
        cx = jnp.sum(jnp.where(onehot, xp, 0.0), axis=1, keepdims=True)
        cy = jnp.sum(jnp.where(onehot, yp, 0.0), axis=1, keepdims=True)
        cz = jnp.sum(jnp.where(onehot, zp, 0.0), axis=1, keepdims=True)
        sel = iota_s == t
        ax = jnp.where(sel, cx, ax)
        ay = jnp.where(sel, cy, ay)
        az = jnp.where(sel, cz, az)
        dx = xp - cx
        dy = yp - cy
        dz = zp - cz
        d = (dx * dx + dy * dy) + dz * dz
        dist = jnp.minimum(dist, d)
        m = jnp.max(dist, axis=1, keepdims=True)
        far = jnp.min(jnp.where(dist == m, iota, N), axis=1, keepdims=True)
        return dist, far, ax, ay, az

    zs = jnp.zeros((B, npoint), jnp.float32)
    init = (jnp.full((B, N), 1e10, jnp.float32),
            jnp.zeros((B, 1), jnp.int32), zs, zs, zs)
    _, _, ax, ay, az = lax.fori_loop(0, npoint, step, init)
    nx_ref[...] = ax
    ny_ref[...] = ay
    nz_ref[...] = az


def _fps(xyz, npoint):
    B, N, _ = xyz.shape
    xt = jnp.transpose(xyz, (0, 2, 1))
    outs = pl.pallas_call(
        functools.partial(_fps_body, npoint=npoint),
        in_specs=[pl.BlockSpec((B, N), lambda: (0, 0))] * 3,
        out_specs=[pl.BlockSpec((B, npoint), lambda: (0, 0))] * 3,
        out_shape=[jax.ShapeDtypeStruct((B, npoint), jnp.float32)] * 3,
    )(xt[:, 0], xt[:, 1], xt[:, 2])
    return jnp.stack(outs, axis=-1)


def _dist_block(a, b2):
    s2 = jnp.sum(a * a, axis=1, keepdims=True)
    d2 = jnp.sum(b2 * b2, axis=1, keepdims=True).reshape(1, -1)
    cr = lax.dot_general(a, b2, (((1,), (1,)), ((), ())),
                         preferred_element_type=jnp.float32)
    return (s2 + d2) - 2.0 * cr


def _dist_body(a_ref, b2_ref, d_ref):
    d_ref[0] = _dist_block(a_ref[0], b2_ref[0])


def _sqdist(src, dst):
    B, S, _ = src.shape
    N = dst.shape[1]
    SB = _div_block(S, 256)
    return pl.pallas_call(
        _dist_body,
        grid=(B, S // SB),
        in_specs=[
            pl.BlockSpec((1, SB, 3), lambda b, s: (b, s, 0)),
            pl.BlockSpec((1, N, 3), lambda b, s: (b, 0, 0)),
        ],
        out_specs=pl.BlockSpec((1, SB, N), lambda b, s: (b, s, 0)),
        out_shape=jax.ShapeDtypeStruct((B, S, N), jnp.float32),
    )(src, dst)


def _sa_tail_body(g_ref, q_ref, w1_ref, b1_ref, g1_ref, be1_ref,
                  w2_ref, b2_ref, g2_ref, be2_ref,
                  w3_ref, b3_ref, g3_ref, be3_ref, o_ref):
    MT, K, Cp = g_ref.shape
    x = g_ref[...] - q_ref[...][:, None, :]
    x = x.reshape(MT * K, Cp)
    x = jnp.dot(x, w1_ref[...], preferred_element_type=jnp.float32) + b1_ref[...]
    x = jax.nn.relu(g1_ref[...] * (x * INV) + be1_ref[...])
    x = jnp.dot(x, w2_ref[...], preferred_element_type=jnp.float32) + b2_ref[...]
    x = jax.nn.relu(g2_ref[...] * (x * INV) + be2_ref[...])
    x = jnp.dot(x, w3_ref[...], preferred_element_type=jnp.float32) + b3_ref[...]
    x = jax.nn.relu(g3_ref[...] * (x * INV) + be3_ref[...])
    C3 = x.shape[1]
    o_ref[...] = jnp.max(x.reshape(MT, K, C3), axis=1)


def _sa_tail(gath, sub, W1p, layers):
    M, K, Cp = gath.shape
    l1, l2, l3 = layers
    C1 = l1['W'].shape[1]
    C2 = l2['W'].shape[1]
    C3 = l3['W'].shape[1]
    MT = _div_block(M, 128)
    return pl.pallas_call(
        _sa_tail_body,
        grid=(M // MT,),
        in_specs=[
            pl.BlockSpec((MT, K, Cp), lambda i: (i, 0, 0)),
            pl.BlockSpec((MT, Cp), lambda i: (i, 0)),
            pl.BlockSpec((Cp, C1), lambda i: (0, 0)),
            pl.BlockSpec((C1,), lambda i: (0,)),
            pl.BlockSpec((C1,), lambda i: (0,)),
            pl.BlockSpec((C1,), lambda i: (0,)),
            pl.BlockSpec((C1, C2), lambda i: (0, 0)),
            pl.BlockSpec((C2,), lambda i: (0,)),
            pl.BlockSpec((C2,), lambda i: (0,)),
            pl.BlockSpec((C2,), lambda i: (0,)),
            pl.BlockSpec((C2, C3), lambda i: (0, 0)),
            pl.BlockSpec((C3,), lambda i: (0,)),
            pl.BlockSpec((C3,), lambda i: (0,)),
            pl.BlockSpec((C3,), lambda i: (0,)),
        ],
        out_specs=pl.BlockSpec((MT, C3), lambda i: (i, 0)),
        out_shape=jax.ShapeDtypeStruct((M, C3), jnp.float32),
    )(gath, sub, W1p, l1['b'], l1['g'], l1['be'],
      l2['W'], l2['b'], l2['g'], l2['be'],
      l3['W'], l3['b'], l3['g'], l3['be'])


def _knn3_body(a_ref, b2_ref, i_ref, w_ref, *, S2):
    a = a_ref[0]
    SB = a.shape[0]
    D = _dist_block(a, b2_ref[0])
    iota = lax.broadcasted_iota(jnp.int32, (SB, S2), 1)
    vals, idxs = [], []
    for _ in range(3):
        m = jnp.min(D, axis=1, keepdims=True)
        am = jnp.min(jnp.where(D == m, iota, S2), axis=1, keepdims=True)
        vals.append(m)
        idxs.append(am)
        D = jnp.where(iota == am, jnp.float32(np.inf), D)
    b = pl.program_id(0)
    i_ref[0] = jnp.concatenate(idxs, axis=1) + b * S2
    d3 = jnp.concatenate(vals, axis=1)
    w = 1.0 / (d3 + 1e-8)
    w_ref[0] = w / jnp.sum(w, axis=1, keepdims=True)


def _knn3(xyz1, xyz2):
    B, S1, _ = xyz1.shape
    S2 = xyz2.shape[1]
    SB = _div_block(S1, 512)
    return pl.pallas_call(
        functools.partial(_knn3_body, S2=S2),
        grid=(B, S1 // SB),
        in_specs=[
            pl.BlockSpec((1, SB, 3), lambda b, s: (b, s, 0)),
            pl.BlockSpec((1, S2, 3), lambda b, s: (b, 0, 0)),
        ],
        out_specs=[
            pl.BlockSpec((1, SB, 3), lambda b, s: (b, s, 0)),
            pl.BlockSpec((1, SB, 3), lambda b, s: (b, s, 0)),
        ],
        out_shape=[
            jax.ShapeDtypeStruct((B, S1, 3), jnp.int32),
            jax.ShapeDtypeStruct((B, S1, 3), jnp.float32),
        ],
    )(xyz1, xyz2)


def _fp_tail_body(*refs, n_layers, has_p1, has_head):
    it = iter(refs)
    g_ref = next(it)
    w_ref = next(it)
    p1_ref = next(it) if has_p1 else None
    lw = [(next(it), next(it), next(it), next(it)) for _ in range(n_layers)]
    if has_head:
        hw1, hb1, hg1, hbe1, hw2, hb2 = (next(it) for _ in range(6))
    o_ref = next(it)

    g = g_ref[...]
    wv = w_ref[...]
    interp = (g[:, 0, :] * wv[:, 0:1] + g[:, 1, :] * wv[:, 1:2]) \
        + g[:, 2, :] * wv[:, 2:3]
    if has_p1:
        x = jnp.concatenate([p1_ref[...], interp], axis=-1)
    else:
        x = interp
    for (W, bb, gg, be) in lw:
        x = jnp.dot(x, W[...], preferred_element_type=jnp.float32) + bb[...]
        x = jax.nn.relu(gg[...] * (x * INV) + be[...])
    if has_head:
        x = jnp.dot(x, hw1[...], preferred_element_type=jnp.float32) + hb1[...]
        x = jax.nn.relu(hg1[...] * (x * INV) + hbe1[...])
        x = jnp.dot(x, hw2[...], preferred_element_type=jnp.float32) + hb2[...]
    o_ref[...] = x


def _fp_tail(gath, w, p1, layers, head=None):
    M, _, C2 = gath.shape
    MT = _div_block(M, 512)
    has_p1 = p1 is not None
    has_head = head is not None
    n_layers = len(layers)

    args = [gath, w]
    in_specs = [
        pl.BlockSpec((MT, 3, C2), lambda i: (i, 0, 0)),
        pl.BlockSpec((MT, 3), lambda i: (i, 0)),
    ]
    cin = C2
    if has_p1:
        C1p = p1.shape[1]
        cin += C1p
        args.append(p1)
        in_specs.append(pl.BlockSpec((MT, C1p), lambda i: (i, 0)))
    c = cin
    for p in layers:
        co = p['W'].shape[1]
        args += [p['W'], p['b'], p['g'], p['be']]
        in_specs += [
            pl.BlockSpec((c, co), lambda i: (0, 0)),
            pl.BlockSpec((co,), lambda i: (0,)),
            pl.BlockSpec((co,), lambda i: (0,)),
            pl.BlockSpec((co,), lambda i: (0,)),
        ]
        c = co
    if has_head:
        hW2, hb2 = head['W2p'], head['b2p']
        co2 = hW2.shape[1]
        args += [head['W1'], head['b1'], head['g1'], head['be1'], hW2, hb2]
        in_specs += [
            pl.BlockSpec((c, 128), lambda i: (0, 0)),
            pl.BlockSpec((128,), lambda i: (0,)),
            pl.BlockSpec((128,), lambda i: (0,)),
            pl.BlockSpec((128,), lambda i: (0,)),
            pl.BlockSpec((128, co2), lambda i: (0, 0)),
            pl.BlockSpec((co2,), lambda i: (0,)),
        ]
        c = co2
    return pl.pallas_call(
        functools.partial(_fp_tail_body, n_layers=n_layers, has_p1=has_p1,
                          has_head=has_head),
        grid=(M // MT,),
        in_specs=in_specs,
        out_specs=pl.BlockSpec((MT, c), lambda i: (i, 0)),
        out_shape=jax.ShapeDtypeStruct((M, c), jnp.float32),
    )(*args)


def _sc_compact_call(D, Nsrc, K1, K2, r1sq, r2sq, B):
    R, N = D.shape
    RW = R // SC_W
    CH = _div_block(RW, 8)
    WPB = SC_W // B
    K1p, K2p = K1 + 16, K2 + 16
    nchunks = N // 16
    r1c = np.float32(r1sq)
    r2c = np.float32(r2sq)
    mesh = plsc.VectorSubcoreMesh(core_axis_name="c", subcore_axis_name="s")

    @functools.partial(
        pl.kernel, mesh=mesh,
        out_type=[
            jax.ShapeDtypeStruct((R, K1p), jnp.int32),
            jax.ShapeDtypeStruct((R, K2p), jnp.int32),
        ],
        scratch_types=[
            pltpu.VMEM((CH, N), jnp.float32),
            pltpu.VMEM((CH, K1p), jnp.int32),
            pltpu.VMEM((CH, K2p), jnp.int32),
        ],
        compiler_params=pltpu.CompilerParams(
            use_tc_tiling_on_sc=False, needs_layout_passes=False),
    )
    def compact(d_hbm, g1_hbm, g2_hbm, dbuf, g1buf, g2buf):
        wid = lax.axis_index("s") * SC_NC + lax.axis_index("c")
        base_pt = (wid // WPB) * N
        row0 = wid * RW

        def outer(i, _):
            rb = row0 + i * CH
            pltpu.sync_copy(d_hbm.at[pl.ds(rb, CH)], dbuf)
            for r in range(CH):
                def cond(carry):
                    c, c1, c2, f1, f2 = carry
                    return (c < nchunks) & ((c1 < K1) | (c2 < K2))

                def body(carry):
                    c, c1, c2, f1, f2 = carry
                    dv = dbuf[r, pl.ds(c * 16, 16)]
                    gi = lax.iota(jnp.int32, 16) + (base_pt + c * 16)
                    m1 = dv <= r1c
                    m2 = dv <= r2c
                    pc1 = plsc.all_reduce_population_count(m1)[0]
                    pc2 = plsc.all_reduce_population_count(m2)[0]
                    off1 = jnp.minimum(c1, K1)
                    off2 = jnp.minimum(c2, K2)
                    plsc.store_compressed(
                        g1buf.at[r].at[pl.ds(off1, 16)], gi, mask=m1)
                    plsc.store_compressed(
                        g2buf.at[r].at[pl.ds(off2, 16)], gi, mask=m2)

                    fm1 = base_pt + c * 16 + plsc.all_reduce_ffs(m1)[0]
                    fm2 = base_pt + c * 16 + plsc.all_reduce_ffs(m2)[0]
                    f1 = jnp.where((c1 == 0) & (pc1 > 0), fm1, f1)
                    f2 = jnp.where((c2 == 0) & (pc2 > 0), fm2, f2)
                    c1 = jnp.where(c1 < K1, c1 + pc1, c1)
                    c2 = jnp.where(c2 < K2, c2 + pc2, c2)
                    return (c + 1, c1, c2, f1, f2)

                last = jnp.int32(base_pt + N - 1)
                init = (jnp.int32(0), jnp.int32(0), jnp.int32(0), last, last)
                _, c1, c2, f1, f2 = lax.while_loop(cond, body, init)
                for j in range(max(1, K1 // 16)):
                    sl = lax.iota(jnp.int32, 16) + j * 16
                    cur = g1buf[r, pl.ds(j * 16, 16)]
                    g1buf[r, pl.ds(j * 16, 16)] = jnp.where(sl < c1, cur, f1)
                for j in range(max(1, K2 // 16)):
                    sl = lax.iota(jnp.int32, 16) + j * 16
                    cur = g2buf[r, pl.ds(j * 16, 16)]
                    g2buf[r, pl.ds(j * 16, 16)] = jnp.where(sl < c2, cur, f2)
            pltpu.sync_copy(g1buf, g1_hbm.at[pl.ds(rb, CH)])
            pltpu.sync_copy(g2buf, g2_hbm.at[pl.ds(rb, CH)])
            return 0

        lax.fori_loop(0, RW // CH, outer, 0)

    g1p, g2p = compact(D)
    return g1p[:, :K1], g2p[:, :K2]


def _sc_gather_call(table, idx):
    V, C = table.shape
    R = idx.shape[0]
    RW = R // SC_W
    CR = 128
    while RW % CR or CR * C * 8 > 262144:
        CR //= 2
    nch = RW // CR
    mesh = plsc.VectorSubcoreMesh(core_axis_name="c", subcore_axis_name="s")

    @functools.partial(
        pl.kernel, mesh=mesh,
        out_type=jax.ShapeDtypeStruct((R, C), jnp.float32),
        scratch_types=[
            pltpu.VMEM((nch, CR), jnp.int32),
            pltpu.VMEM((CR, C), jnp.float32),
            pltpu.VMEM((CR, C), jnp.float32),
            pltpu.SemaphoreType.DMA,
            pltpu.SemaphoreType.DMA,
        ],
        compiler_params=pltpu.CompilerParams(use_tc_tiling_on_sc=False),
    )
    def gather(tab_hbm, idx_hbm, out_hbm, idxv, rows0, rows1, sem0, sem1):
        wid = lax.axis_index("s") * SC_NC + lax.axis_index("c")
        pltpu.sync_copy(idx_hbm.at[pl.ds(wid * nch, nch)], idxv)
        rows = (rows0, rows1)
        sems = (sem0, sem1)
        hs = [None, None]
        hs[0] = pltpu.async_copy(tab_hbm.at[idxv.at[0]], rows0, sem0)
        for i in range(nch):
            cur = i % 2
            if i + 1 < nch:
                nxt = (i + 1) % 2
                hs[nxt] = pltpu.async_copy(
                    tab_hbm.at[idxv.at[i + 1]], rows[nxt], sems[nxt])
            hs[cur].wait()
            pltpu.sync_copy(rows[cur], out_hbm.at[pl.ds(wid * RW + i * CR, CR)])

    return gather(table, idx.reshape(R // CR, CR))


def _sa_msg_layer(xyz, points, npoint, radii, nsamples, branches):
    B, N, _ = xyz.shape
    new_xyz = _fps(xyz, npoint)
    D = _sqdist(new_xyz, xyz).reshape(B * npoint, N)
    (r1, r2), (K1, K2) = radii, nsamples
    g1, g2 = _sc_compact_call(D, N, K1, K2, r1 * r1, r2 * r2, B)

    C = points.shape[-1]
    Cin = C + 3
    Cp = ((Cin + 15) // 16) * 16
    M = B * npoint
    X = jnp.concatenate([points, xyz], axis=-1).reshape(B * N, Cin)
    Xp = jnp.pad(X, ((0, 0), (0, Cp - Cin)))
    sub = jnp.pad(new_xyz.reshape(M, 3), ((0, 0), (C, Cp - Cin)))
    outs = []
    for (gidx, K, layers) in ((g1, K1, branches[0]), (g2, K2, branches[1])):
        W1p = jnp.pad(layers[0]['W'], ((0, Cp - Cin), (0, 0)))
        G = _sc_gather_call(Xp, gidx.reshape(-1))
        G = G.reshape(M, K, Cp)
        outs.append(_sa_tail(G, sub, W1p, layers))
    new_points = jnp.concatenate(outs, axis=-1).reshape(B, npoint, -1)
    return new_xyz, new_points


def _fp_layer(xyz1, xyz2, points1, points2, layers, head=None):
    B, S1, _ = xyz1.shape
    S2 = xyz2.shape[1]
    C2 = points2.shape[-1]
    idx, w = _knn3(xyz1, xyz2)
    G = _sc_gather_call(points2.reshape(B * S2, C2), idx.reshape(-1))
    G = G.reshape(B * S1, 3, C2)
    wf = w.reshape(B * S1, 3)
    p1 = None if points1 is None else points1.reshape(B * S1, -1)
    out = _fp_tail(G, wf, p1, layers, head=head)
    return out.reshape(B, S1, out.shape[-1])


def _forward(input_pc, params, sa_cfgs, num_classes):
    B = input_pc.shape[0]
    xyz0 = input_pc[:, :, :3]
    feat0 = input_pc[:, :, 3:]
    h = params['head']
    head = dict(W1=h['W1'], b1=h['b1'], g1=h['g1'], be1=h['be1'],
                W2p=jnp.zeros((128, 128), jnp.float32).at[:, :num_classes].set(h['W2']),
                b2p=jnp.zeros((128,), jnp.float32).at[:num_classes].set(h['b2']))

    xs, ps = [xyz0], [feat0]
    x, p = xyz0, feat0
    for cfg, br in zip(sa_cfgs, params['sa']):
        npoint, radii, nsamples = cfg[0], cfg[1], cfg[2]
        x, p = _sa_msg_layer(x, p, npoint, radii, nsamples, br)
        xs.append(x)
        ps.append(p)

    n = len(xs) - 1
    for i in range(n - 1):
        ps[n - 1 - i] = _fp_layer(xs[n - 1 - i], xs[n - i], ps[n - 1 - i],
                                  ps[n - i], params['fp'][i])
    out = _fp_layer(xs[0], xs[1], None, ps[1], params['fp'][n - 1], head=head)
    return out[:, :, :num_classes]


_SA_CFGS = [
    (1024, [1.0, 3.0], [8, 32]),
    (512, [2.0, 4.0], [8, 32]),
    (256, [3.0, 6.0], [16, 32]),
    (128, [4.0, 8.0], [16, 32]),
]


def kernel(input_pc, params):
    return _forward(input_pc, params, _SA_CFGS, 13)
